# Initial kernel scaffold; baseline (speedup 1.0000x reference)
import functools

import jax
import jax.numpy as jnp
from jax import lax
from jax.experimental import pallas as pl

_N_S = 4
_N_LR = 22
_K = 40


def _gravnet_body(x_ref, xT_ref, WsT_ref, Ws_ref, bs_row_ref, bs_col_ref,
                  WoT_ref, bo_ref, out_ref):
    V = x_ref.shape[1]
    xb = x_ref[0]
    xbT = xT_ref[0]

    mean_row = jnp.mean(xb, axis=0, keepdims=True)
    x2 = jnp.concatenate([xb, jnp.broadcast_to(mean_row, xb.shape)], axis=1)
    mean_col = jnp.mean(xbT, axis=1, keepdims=True)
    x2T = jnp.concatenate([xbT, jnp.broadcast_to(mean_col, xbT.shape)], axis=0)

    hi_p = jax.lax.Precision.HIGHEST
    slr = jnp.maximum(
        jnp.dot(x2, WsT_ref[...], preferred_element_type=jnp.float32,
                precision=hi_p) + bs_row_ref[...], 0.0)
    slrT = jnp.maximum(
        jnp.dot(Ws_ref[...], x2T, preferred_element_type=jnp.float32,
                precision=hi_p) + bs_col_ref[...], 0.0)

    def coord_sq(c):
        d = slr[:, c:c + 1] - slrT[c:c + 1, :]
        return d * d
    sq = (coord_sq(0) + coord_sq(1)) + (coord_sq(2) + coord_sq(3))

    bits = jax.lax.bitcast_convert_type(sq, jnp.int32)
    lo0 = jnp.zeros((V, 1), jnp.int32)
    hi0 = jnp.max(bits, axis=1, keepdims=True)
    kf = jnp.float32(_K)

    def bs_step(_, carry):
        lo, hi = carry
        mid = lo + ((hi - lo) >> 1)
        cnt = jnp.sum((bits <= mid).astype(jnp.float32), axis=1, keepdims=True)
        ge = cnt >= kf
        return jnp.where(ge, lo, mid + 1), jnp.where(ge, mid, hi)

    lo, hi = lax.fori_loop(0, 31, bs_step, (lo0, hi0))
    tstar = lo

    L = (bits < tstar).astype(jnp.float32)
    E = (bits == tstar).astype(jnp.float32)
    c1 = jnp.sum(L, axis=1, keepdims=True)
    m = kf - c1

    jj = lax.broadcasted_iota(jnp.int32, (V, V), 0)
    kk = lax.broadcasted_iota(jnp.int32, (V, V), 1)
    tri = (jj < kk).astype(jnp.float32)
    R = jnp.dot(E, tri, preferred_element_type=jnp.float32)
    sel = L + E * (R < m).astype(jnp.float32)

    d = jnp.sqrt(sq)
    wgt = jnp.exp(-10.0 * d * d)
    A = sel * wgt

    lr = slr[:, _N_S:]
    mean_ft = jnp.dot(A, lr, preferred_element_type=jnp.float32,
                      precision=hi_p) * (1.0 / _K)

    cols = []
    for c in range(_N_LR):
        prod = A * slrT[_N_S + c:_N_S + c + 1, :]
        cols.append(jnp.max(prod, axis=1, keepdims=True))
    max_ft = jnp.concatenate(cols, axis=1)

    fp = jnp.concatenate([x2, mean_ft, max_ft], axis=1)
    out = jnp.maximum(
        jnp.dot(fp, WoT_ref[...], preferred_element_type=jnp.float32,
                precision=hi_p) + bo_ref[...], 0.0)
    out_ref[0] = out


def kernel(x, W_slr, b_slr, W_out, b_out):
    B, V, F = x.shape
    n_out = W_out.shape[0]
    xT = jnp.swapaxes(x, 1, 2)

    out = pl.pallas_call(
        _gravnet_body,
        grid=(B,),
        in_specs=[
            pl.BlockSpec((1, V, F), lambda b: (b, 0, 0)),
            pl.BlockSpec((1, F, V), lambda b: (b, 0, 0)),
            pl.BlockSpec(W_slr.shape[::-1], lambda b: (0, 0)),
            pl.BlockSpec(W_slr.shape, lambda b: (0, 0)),
            pl.BlockSpec((1, W_slr.shape[0]), lambda b: (0, 0)),
            pl.BlockSpec((W_slr.shape[0], 1), lambda b: (0, 0)),
            pl.BlockSpec(W_out.shape[::-1], lambda b: (0, 0)),
            pl.BlockSpec((1, n_out), lambda b: (0, 0)),
        ],
        out_specs=pl.BlockSpec((1, V, n_out), lambda b: (b, 0, 0)),
        out_shape=jax.ShapeDtypeStruct((B, V, n_out), jnp.float32),
    )(x, xT, W_slr.T, W_slr, b_slr.reshape(1, -1), b_slr.reshape(-1, 1),
      W_out.T, b_out.reshape(1, -1))
    return out


# speedup vs baseline: 5.5232x; 5.5232x over previous
import functools

import jax
import jax.numpy as jnp
from jax import lax
from jax.experimental import pallas as pl

_N_S = 4
_N_LR = 22
_K = 40


def _gravnet_body(x_ref, WsT_ref, bs_row_ref, WoT_ref, bo_ref, out_ref):
    V = x_ref.shape[1]
    xb = x_ref[0]

    mean_row = jnp.mean(xb, axis=0, keepdims=True)
    x2 = jnp.concatenate([xb, jnp.broadcast_to(mean_row, xb.shape)], axis=1)

    hi_p = jax.lax.Precision.HIGHEST
    slr = jnp.maximum(
        jnp.dot(x2, WsT_ref[...],
                preferred_element_type=jnp.float32) + bs_row_ref[...], 0.0)
    slrT = slr.T

    def coord_sq(c):
        d = slr[:, c:c + 1] - slrT[c:c + 1, :]
        return d * d
    sq = (coord_sq(0) + coord_sq(1)) + (coord_sq(2) + coord_sq(3))

    d = jnp.sqrt(sq)
    bits = jax.lax.bitcast_convert_type(d, jnp.int32)
    lo0 = jnp.zeros((V, 1), jnp.int32)
    hi0 = jnp.max(bits, axis=1, keepdims=True)
    kf = jnp.float32(_K)

    def bs_step(_, carry):
        lo, hi = carry
        mid = lo + ((hi - lo) >> 1)
        cnt = jnp.sum((bits <= mid).astype(jnp.float32), axis=1, keepdims=True)
        ge = cnt >= kf
        return jnp.where(ge, lo, mid + 1), jnp.where(ge, mid, hi)

    lo, hi = lax.fori_loop(0, 31, bs_step, (lo0, hi0))
    tstar = lo

    L = (bits < tstar).astype(jnp.float32)
    E = (bits == tstar).astype(jnp.float32)
    c1 = jnp.sum(L, axis=1, keepdims=True)
    m = kf - c1

    jj = lax.broadcasted_iota(jnp.int32, (V, V), 0)
    kk = lax.broadcasted_iota(jnp.int32, (V, V), 1)
    tri = (jj < kk).astype(jnp.float32)
    R = jnp.dot(E, tri, preferred_element_type=jnp.float32)
    sel = L + E * (R < m).astype(jnp.float32)

    wgt = jnp.exp(-10.0 * d * d)
    A = sel * wgt

    lr = slr[:, _N_S:]
    mean_ft = jnp.dot(A, lr, preferred_element_type=jnp.float32,
                      precision=hi_p) * (1.0 / _K)

    cols = []
    for c in range(_N_LR):
        prod = A * slrT[_N_S + c:_N_S + c + 1, :]
        cols.append(jnp.max(prod, axis=1, keepdims=True))
    max_ft = jnp.concatenate(cols, axis=1)

    fp = jnp.concatenate([x2, mean_ft, max_ft], axis=1)
    out = jnp.maximum(
        jnp.dot(fp, WoT_ref[...],
                preferred_element_type=jnp.float32) + bo_ref[...], 0.0)
    out_ref[0] = out


def kernel(x, W_slr, b_slr, W_out, b_out):
    B, V, F = x.shape
    n_out = W_out.shape[0]

    out = pl.pallas_call(
        _gravnet_body,
        grid=(B,),
        in_specs=[
            pl.BlockSpec((1, V, F), lambda b: (b, 0, 0)),
            pl.BlockSpec(W_slr.shape[::-1], lambda b: (0, 0)),
            pl.BlockSpec((1, W_slr.shape[0]), lambda b: (0, 0)),
            pl.BlockSpec(W_out.shape[::-1], lambda b: (0, 0)),
            pl.BlockSpec((1, n_out), lambda b: (0, 0)),
        ],
        out_specs=pl.BlockSpec((1, V, n_out), lambda b: (b, 0, 0)),
        out_shape=jax.ShapeDtypeStruct((B, V, n_out), jnp.float32),
    )(x, W_slr.T, b_slr.reshape(1, -1), W_out.T, b_out.reshape(1, -1))
    return out
